# baseline (speedup 1.0000x reference)
import dataclasses
import functools

import jax
import jax.numpy as jnp
from jax import lax
from jax.experimental import pallas as pl
from jax.experimental.pallas import tpu as pltpu
from jax.experimental.pallas import tpu_sc as plsc

N_NODES = 10000
D_IN = 128
D_OUT = 256
E = 160000
N_CLS = 1000

NC, NS, LANES = 2, 16, 16
NW = NC * NS
ECH = 96
ENCH = 53
EPAD = NW * ENCH * ECH
NACC = 10240
ROWS_PER_SUB = NACC // NS
ZR = 64

CH = 128
C_CODES = 16 * 64 * 32
CODE_NCH = C_CODES // (NW * CH)

@functools.cache
def _vector_mesh():
    return plsc.VectorSubcoreMesh(core_axis_name="c", subcore_axis_name="s",
                                  num_cores=NC, num_subcores=NS)


_sc_params = pltpu.CompilerParams()
if "needs_layout_passes" in pltpu.CompilerParams.__dataclass_fields__:
    _sc_params = dataclasses.replace(_sc_params, needs_layout_passes=False)


def _wid():
    return lax.axis_index("s") * NC + lax.axis_index("c")


def _zero_acc(zeros_v, acc):
    z = jnp.zeros((LANES,), jnp.float32)

    @pl.loop(0, ZR)
    def _(r):
        for f in range(8):
            zeros_v[r, pl.ds(16 * f, 16)] = z

    s = lax.axis_index("s")

    @pl.loop(0, ROWS_PER_SUB // ZR)
    def _(k):
        pltpu.sync_copy(zeros_v.at[pl.ds(0, ZR)],
                        acc.at[pl.ds(s * ROWS_PER_SUB + k * ZR, ZR)])


def _scale_rows(rows_v, w_v, c):
    c16 = jnp.broadcast_to(c, (LANES,))

    @plsc.parallel_loop(0, ECH, 2, unroll=2)
    def _(i):
        w0 = plsc.load_gather(w_v, [c16, jnp.broadcast_to(i, (LANES,))])
        w1 = plsc.load_gather(w_v, [c16, jnp.broadcast_to(i + 1, (LANES,))])
        for f in range(8):
            sl0 = (i, pl.ds(16 * f, 16))
            sl1 = (i + 1, pl.ds(16 * f, 16))
            a0 = rows_v[sl0]
            a1 = rows_v[sl1]
            rows_v[sl0] = a0 * w0
            rows_v[sl1] = a1 * w1


def _segsum_body(table, srcp, dstp, wp, out0, out1,
                 src_v, dst_v, w_v, rows_a, rows_b, sem_a, sem_b,
                 acc):
    wid = _wid()
    core = lax.axis_index("c")
    s = lax.axis_index("s")

    _zero_acc(rows_a, acc)
    pltpu.sync_copy(srcp.at[wid], src_v)
    pltpu.sync_copy(dstp.at[wid], dst_v)
    pltpu.sync_copy(wp.at[wid], w_v)
    plsc.subcore_barrier()

    pltpu.async_copy(table.at[src_v.at[0]], rows_a, sem_a)

    @pl.loop(0, ENCH // 2)
    def _(k):
        c0 = 2 * k
        pltpu.async_copy(table.at[src_v.at[c0 + 1]], rows_b, sem_b)
        pltpu.make_async_copy(table.at[src_v.at[c0]], rows_a, sem_a).wait()
        _scale_rows(rows_a, w_v, c0)
        pltpu.sync_copy(rows_a, acc.at[dst_v.at[c0]], add=True)
        pltpu.async_copy(table.at[src_v.at[c0 + 2]], rows_a, sem_a)
        pltpu.make_async_copy(table.at[src_v.at[c0 + 1]], rows_b, sem_b).wait()
        _scale_rows(rows_b, w_v, c0 + 1)
        pltpu.sync_copy(rows_b, acc.at[dst_v.at[c0 + 1]], add=True)

    pltpu.make_async_copy(table.at[src_v.at[ENCH - 1]], rows_a, sem_a).wait()
    _scale_rows(rows_a, w_v, ENCH - 1)
    pltpu.sync_copy(rows_a, acc.at[dst_v.at[ENCH - 1]], add=True)

    plsc.subcore_barrier()

    out = [out0, out1]

    @pl.loop(0, ROWS_PER_SUB // ZR)
    def _(k):
        rs = pl.ds(s * ROWS_PER_SUB + k * ZR, ZR)
        for ci in range(NC):
            @pl.when(core == ci)
            def _():
                pltpu.sync_copy(acc.at[rs], out[ci].at[rs])


@jax.jit
def _sc_segsum(table, srcp, dstp, wp):
    f32 = jnp.float32
    k = pl.kernel(
        _segsum_body,
        out_type=(jax.ShapeDtypeStruct((NACC, D_IN), f32),
                  jax.ShapeDtypeStruct((NACC, D_IN), f32)),
        mesh=_vector_mesh(),
        scratch_types=[
            pltpu.VMEM((ENCH, ECH), jnp.int32),
            pltpu.VMEM((ENCH, ECH), jnp.int32),
            pltpu.VMEM((ENCH, ECH), f32),
            pltpu.VMEM((ECH, D_IN), f32),
            pltpu.VMEM((ECH, D_IN), f32),
            pltpu.SemaphoreType.DMA,
            pltpu.SemaphoreType.DMA,
            pltpu.VMEM_SHARED((NACC, D_IN), f32),
        ],
        compiler_params=_sc_params,
    )
    return k(table, srcp, dstp, wp)


def _code_gather_body(tab, codes, g, idx_v, rows_a, rows_b, sem_a, sem_b):
    wid = _wid()
    base = wid * (CODE_NCH * CH)
    pltpu.sync_copy(codes.at[wid], idx_v)
    pltpu.async_copy(tab.at[idx_v.at[0]], rows_a, sem_a)

    @pl.loop(0, CODE_NCH // 2)
    def _(k):
        c0 = 2 * k
        pltpu.async_copy(tab.at[idx_v.at[c0 + 1]], rows_b, sem_b)
        pltpu.make_async_copy(tab.at[idx_v.at[c0]], rows_a, sem_a).wait()
        pltpu.sync_copy(rows_a, g.at[pl.ds(base + c0 * CH, CH)])

        @pl.when(k < CODE_NCH // 2 - 1)
        def _():
            pltpu.async_copy(tab.at[idx_v.at[c0 + 2]], rows_a, sem_a)

        pltpu.make_async_copy(tab.at[idx_v.at[c0 + 1]], rows_b, sem_b).wait()
        pltpu.sync_copy(rows_b, g.at[pl.ds(base + (c0 + 1) * CH, CH)])


@jax.jit
def _sc_code_gather(tab, codes):
    f32 = jnp.float32
    k = pl.kernel(
        _code_gather_body,
        out_type=jax.ShapeDtypeStruct((C_CODES, D_IN), f32),
        mesh=_vector_mesh(),
        scratch_types=[
            pltpu.VMEM((CODE_NCH, CH), jnp.int32),
            pltpu.VMEM((CH, D_IN), f32),
            pltpu.VMEM((CH, D_IN), f32),
            pltpu.SemaphoreType.DMA,
            pltpu.SemaphoreType.DMA,
        ],
        compiler_params=_sc_params,
    )
    return k(tab, codes)


def _mm2_relu_body(a_ref, b_ref, w_ref, bias_ref, o_ref):
    acc = jnp.dot(a_ref[...] + b_ref[...], w_ref[...],
                  preferred_element_type=jnp.float32)
    o_ref[...] = jnp.maximum(acc + bias_ref[...], 0.0)


def _mm2_relu(a, b, w, bias, blk):
    n, d = a.shape
    dout = w.shape[1]
    grid = pl.cdiv(n, blk)
    return pl.pallas_call(
        _mm2_relu_body,
        grid=(grid,),
        in_specs=[
            pl.BlockSpec((blk, d), lambda i: (i, 0)),
            pl.BlockSpec((blk, d), lambda i: (i, 0)),
            pl.BlockSpec((d, dout), lambda i: (0, 0)),
            pl.BlockSpec((1, dout), lambda i: (0, 0)),
        ],
        out_specs=pl.BlockSpec((blk, dout), lambda i: (i, 0)),
        out_shape=jax.ShapeDtypeStruct((n, dout), jnp.float32),
    )(a, b, w, bias.reshape(1, -1))


CB = 4096


def _code_attn_body(g_ref, w2_ref, b2_ref, ac_ref, cf_ref, o_ref):
    f32 = jnp.float32
    g = g_ref[...]
    zT = lax.dot_general(w2_ref[...], g, (((0,), (1,)), ((), ())),
                         preferred_element_type=f32)
    ones = jnp.ones((1, CB), f32)
    biasT = lax.dot_general(b2_ref[...], ones, (((0,), (0,)), ((), ())),
                            preferred_element_type=f32)
    ceT = jnp.maximum(zT + biasT, 0.0)
    s = jnp.dot(ac_ref[...], ceT, preferred_element_type=f32)
    mask = cf_ref[0] > 0.0
    es = jnp.where(mask, jnp.exp(s), 0.0)
    r = lax.broadcasted_iota(jnp.int32, (CB, 128), 0)
    v = lax.broadcasted_iota(jnp.int32, (CB, 128), 1)
    bd = ((r // 32) == v).astype(f32)
    num = jnp.dot(ceT * es, bd, preferred_element_type=f32)
    den = jnp.dot(es, bd, preferred_element_type=f32)
    o_ref[...] = num / jnp.maximum(den, 1e-30)


def _visit_attn_decoder_body(vt_ref, vm_ref, av_ref, w1_ref, b1_ref,
                             w2_ref, b2_ref, w3_ref, b3_ref, out_ref):
    f32 = jnp.float32
    vT = vt_ref[...]
    vs = jnp.dot(av_ref[...], vT, preferred_element_type=f32)
    es = jnp.where(vm_ref[...] > 0.0, jnp.exp(vs), 0.0)
    r = lax.broadcasted_iota(jnp.int32, (1024, 16), 0)
    b = lax.broadcasted_iota(jnp.int32, (1024, 16), 1)
    bd = ((r // 64) == b).astype(f32)
    num = jnp.dot(vT * es, bd, preferred_element_type=f32)
    den = jnp.dot(es, bd, preferred_element_type=f32)
    pT = num / jnp.maximum(den, 1e-30)
    i = lax.broadcasted_iota(jnp.int32, (16, 16), 0)
    j = lax.broadcasted_iota(jnp.int32, (16, 16), 1)
    eye = (i == j).astype(f32)
    pe = lax.dot_general(eye, pT, (((1,), (1,)), ((), ())),
                         preferred_element_type=f32)
    h1 = jnp.maximum(
        jnp.dot(pe, w1_ref[...], preferred_element_type=f32) + b1_ref[...],
        0.0)
    h2 = jnp.maximum(
        jnp.dot(h1, w2_ref[...], preferred_element_type=f32) + b2_ref[...],
        0.0)
    out_ref[...] = (
        jnp.dot(h2, w3_ref[...], preferred_element_type=f32) + b3_ref[...])


def kernel(edge_index, edge_weight, codes_x, node_emb, gcn_W1, gcn_b1,
           gcn_W2, gcn_b2, attn_code, attn_visit, dec_W1, dec_b1,
           dec_W2, dec_b2, dec_W3, dec_b3):
    f32 = jnp.float32
    pad = EPAD - E
    spread = (jnp.arange(pad, dtype=jnp.int32) * 131) % N_NODES
    srcp = jnp.concatenate([edge_index[0], spread]).reshape(NW, ENCH, ECH)
    dstp = jnp.concatenate([edge_index[1], spread]).reshape(NW, ENCH, ECH)
    wp = jnp.concatenate([edge_weight, jnp.zeros((pad,), f32)]
                         ).reshape(NW, ENCH, ECH)

    p10, p11 = _sc_segsum(node_emb, srcp, dstp, wp)
    h = _mm2_relu(p10, p11, gcn_W1, gcn_b1, 2048)

    p20, p21 = _sc_segsum(h, srcp, dstp, wp)
    a2 = p20 + p21

    codes = codes_x.reshape(NW, CODE_NCH, CH)
    g = _sc_code_gather(a2, codes)

    codesf = codes_x.astype(f32).reshape(C_CODES // CB, 1, CB)
    visit_T = pl.pallas_call(
        _code_attn_body,
        grid=(C_CODES // CB,),
        in_specs=[
            pl.BlockSpec((CB, D_IN), lambda i: (i, 0)),
            pl.BlockSpec((D_IN, D_OUT), lambda i: (0, 0)),
            pl.BlockSpec((1, D_OUT), lambda i: (0, 0)),
            pl.BlockSpec((1, D_OUT), lambda i: (0, 0)),
            pl.BlockSpec((1, 1, CB), lambda i: (i, 0, 0)),
        ],
        out_specs=pl.BlockSpec((D_OUT, CB // 32), lambda i: (0, i)),
        out_shape=jax.ShapeDtypeStruct((D_OUT, 1024), f32),
    )(g, gcn_W2, gcn_b2.reshape(1, -1), attn_code, codesf)

    vmask = jnp.any(codes_x > 0, axis=-1).astype(f32).reshape(1, 1024)
    w3p = jnp.zeros((dec_W3.shape[0], 1024), f32).at[:, :N_CLS].set(dec_W3)
    b3p = jnp.zeros((1, 1024), f32).at[0, :N_CLS].set(dec_b3)
    out = pl.pallas_call(
        _visit_attn_decoder_body,
        out_shape=jax.ShapeDtypeStruct((16, 1024), f32),
    )(visit_T, vmask, attn_visit, dec_W1, dec_b1.reshape(1, -1), dec_W2,
      dec_b2.reshape(1, -1), w3p, b3p)
    return out[:, :N_CLS]


# speedup vs baseline: 1.0046x; 1.0046x over previous
import dataclasses
import functools

import jax
import jax.numpy as jnp
from jax import lax
from jax.experimental import pallas as pl
from jax.experimental.pallas import tpu as pltpu
from jax.experimental.pallas import tpu_sc as plsc

N_NODES = 10000
D_IN = 128
D_OUT = 256
E = 160000
N_CLS = 1000

NC, NS, LANES = 2, 16, 16
NW = NC * NS
ECH = 96
ENCH = 53
EPAD = NW * ENCH * ECH
NACC = 10240
ROWS_PER_SUB = NACC // NS
ZR = 64

CH = 128
C_CODES = 16 * 64 * 32
CODE_NCH = C_CODES // (NW * CH)

@functools.cache
def _vector_mesh():
    return plsc.VectorSubcoreMesh(core_axis_name="c", subcore_axis_name="s",
                                  num_cores=NC, num_subcores=NS)


_sc_params = pltpu.CompilerParams()
if "needs_layout_passes" in pltpu.CompilerParams.__dataclass_fields__:
    _sc_params = dataclasses.replace(_sc_params, needs_layout_passes=False)


def _wid():
    return lax.axis_index("s") * NC + lax.axis_index("c")


def _zero_acc(zeros_v, acc):
    z = jnp.zeros((LANES,), jnp.float32)

    @pl.loop(0, ZR)
    def _(r):
        for f in range(8):
            zeros_v[r, pl.ds(16 * f, 16)] = z

    s = lax.axis_index("s")

    @pl.loop(0, ROWS_PER_SUB // ZR)
    def _(k):
        pltpu.sync_copy(zeros_v.at[pl.ds(0, ZR)],
                        acc.at[pl.ds(s * ROWS_PER_SUB + k * ZR, ZR)])


def _scale_rows(rows_v, w_v, c):
    c16 = jnp.broadcast_to(c, (LANES,))

    @plsc.parallel_loop(0, ECH, 1, unroll=4)
    def _(i):
        wreg = plsc.load_gather(w_v, [c16, jnp.broadcast_to(i, (LANES,))])
        for f in range(8):
            sl = (i, pl.ds(16 * f, 16))
            rows_v[sl] = rows_v[sl] * wreg


def _segsum_body(table, srcp, dstp, wp, out0, out1,
                 src_v, dst_v, w_v, rows_a, rows_b, sem_a, sem_b,
                 acc):
    wid = _wid()
    core = lax.axis_index("c")
    s = lax.axis_index("s")

    _zero_acc(rows_a, acc)
    pltpu.sync_copy(srcp.at[wid], src_v)
    pltpu.sync_copy(dstp.at[wid], dst_v)
    pltpu.sync_copy(wp.at[wid], w_v)
    plsc.subcore_barrier()

    pltpu.async_copy(table.at[src_v.at[0]], rows_a, sem_a)

    @pl.loop(0, ENCH // 2)
    def _(k):
        c0 = 2 * k
        pltpu.async_copy(table.at[src_v.at[c0 + 1]], rows_b, sem_b)
        pltpu.make_async_copy(table.at[src_v.at[c0]], rows_a, sem_a).wait()
        _scale_rows(rows_a, w_v, c0)
        pltpu.sync_copy(rows_a, acc.at[dst_v.at[c0]], add=True)
        pltpu.async_copy(table.at[src_v.at[c0 + 2]], rows_a, sem_a)
        pltpu.make_async_copy(table.at[src_v.at[c0 + 1]], rows_b, sem_b).wait()
        _scale_rows(rows_b, w_v, c0 + 1)
        pltpu.sync_copy(rows_b, acc.at[dst_v.at[c0 + 1]], add=True)

    pltpu.make_async_copy(table.at[src_v.at[ENCH - 1]], rows_a, sem_a).wait()
    _scale_rows(rows_a, w_v, ENCH - 1)
    pltpu.sync_copy(rows_a, acc.at[dst_v.at[ENCH - 1]], add=True)

    plsc.subcore_barrier()

    out = [out0, out1]

    @pl.loop(0, ROWS_PER_SUB // ZR)
    def _(k):
        rs = pl.ds(s * ROWS_PER_SUB + k * ZR, ZR)
        for ci in range(NC):
            @pl.when(core == ci)
            def _():
                pltpu.sync_copy(acc.at[rs], out[ci].at[rs])


@jax.jit
def _sc_segsum(table, srcp, dstp, wp):
    f32 = jnp.float32
    k = pl.kernel(
        _segsum_body,
        out_type=(jax.ShapeDtypeStruct((NACC, D_IN), f32),
                  jax.ShapeDtypeStruct((NACC, D_IN), f32)),
        mesh=_vector_mesh(),
        scratch_types=[
            pltpu.VMEM((ENCH, ECH), jnp.int32),
            pltpu.VMEM((ENCH, ECH), jnp.int32),
            pltpu.VMEM((ENCH, ECH), f32),
            pltpu.VMEM((ECH, D_IN), f32),
            pltpu.VMEM((ECH, D_IN), f32),
            pltpu.SemaphoreType.DMA,
            pltpu.SemaphoreType.DMA,
            pltpu.VMEM_SHARED((NACC, D_IN), f32),
        ],
        compiler_params=_sc_params,
    )
    return k(table, srcp, dstp, wp)


def _code_gather_body(tab, codes, g, idx_v, rows_a, rows_b, sem_a, sem_b):
    wid = _wid()
    base = wid * (CODE_NCH * CH)
    pltpu.sync_copy(codes.at[wid], idx_v)
    pltpu.async_copy(tab.at[idx_v.at[0]], rows_a, sem_a)

    @pl.loop(0, CODE_NCH // 2)
    def _(k):
        c0 = 2 * k
        pltpu.async_copy(tab.at[idx_v.at[c0 + 1]], rows_b, sem_b)
        pltpu.make_async_copy(tab.at[idx_v.at[c0]], rows_a, sem_a).wait()
        pltpu.sync_copy(rows_a, g.at[pl.ds(base + c0 * CH, CH)])

        @pl.when(k < CODE_NCH // 2 - 1)
        def _():
            pltpu.async_copy(tab.at[idx_v.at[c0 + 2]], rows_a, sem_a)

        pltpu.make_async_copy(tab.at[idx_v.at[c0 + 1]], rows_b, sem_b).wait()
        pltpu.sync_copy(rows_b, g.at[pl.ds(base + (c0 + 1) * CH, CH)])


@jax.jit
def _sc_code_gather(tab, codes):
    f32 = jnp.float32
    k = pl.kernel(
        _code_gather_body,
        out_type=jax.ShapeDtypeStruct((C_CODES, D_IN), f32),
        mesh=_vector_mesh(),
        scratch_types=[
            pltpu.VMEM((CODE_NCH, CH), jnp.int32),
            pltpu.VMEM((CH, D_IN), f32),
            pltpu.VMEM((CH, D_IN), f32),
            pltpu.SemaphoreType.DMA,
            pltpu.SemaphoreType.DMA,
        ],
        compiler_params=_sc_params,
    )
    return k(tab, codes)


def _mm2_relu_body(a_ref, b_ref, w_ref, bias_ref, o_ref):
    acc = jnp.dot(a_ref[...] + b_ref[...], w_ref[...],
                  preferred_element_type=jnp.float32)
    o_ref[...] = jnp.maximum(acc + bias_ref[...], 0.0)


def _mm2_relu(a, b, w, bias, blk):
    n, d = a.shape
    dout = w.shape[1]
    grid = pl.cdiv(n, blk)
    return pl.pallas_call(
        _mm2_relu_body,
        grid=(grid,),
        in_specs=[
            pl.BlockSpec((blk, d), lambda i: (i, 0)),
            pl.BlockSpec((blk, d), lambda i: (i, 0)),
            pl.BlockSpec((d, dout), lambda i: (0, 0)),
            pl.BlockSpec((1, dout), lambda i: (0, 0)),
        ],
        out_specs=pl.BlockSpec((blk, dout), lambda i: (i, 0)),
        out_shape=jax.ShapeDtypeStruct((n, dout), jnp.float32),
    )(a, b, w, bias.reshape(1, -1))


CB = 4096


def _code_attn_body(g_ref, w2_ref, b2_ref, ac_ref, cf_ref, o_ref):
    f32 = jnp.float32
    g = g_ref[...]
    zT = lax.dot_general(w2_ref[...], g, (((0,), (1,)), ((), ())),
                         preferred_element_type=f32)
    ones = jnp.ones((1, CB), f32)
    biasT = lax.dot_general(b2_ref[...], ones, (((0,), (0,)), ((), ())),
                            preferred_element_type=f32)
    ceT = jnp.maximum(zT + biasT, 0.0)
    s = jnp.dot(ac_ref[...], ceT, preferred_element_type=f32)
    mask = cf_ref[0] > 0.0
    es = jnp.where(mask, jnp.exp(s), 0.0)
    r = lax.broadcasted_iota(jnp.int32, (CB, 128), 0)
    v = lax.broadcasted_iota(jnp.int32, (CB, 128), 1)
    bd = ((r // 32) == v).astype(f32)
    num = jnp.dot(ceT * es, bd, preferred_element_type=f32)
    den = jnp.dot(es, bd, preferred_element_type=f32)
    o_ref[...] = num / jnp.maximum(den, 1e-30)


def _visit_attn_decoder_body(vt_ref, vm_ref, av_ref, w1_ref, b1_ref,
                             w2_ref, b2_ref, w3_ref, b3_ref, out_ref):
    f32 = jnp.float32
    vT = vt_ref[...]
    vs = jnp.dot(av_ref[...], vT, preferred_element_type=f32)
    es = jnp.where(vm_ref[...] > 0.0, jnp.exp(vs), 0.0)
    r = lax.broadcasted_iota(jnp.int32, (1024, 16), 0)
    b = lax.broadcasted_iota(jnp.int32, (1024, 16), 1)
    bd = ((r // 64) == b).astype(f32)
    num = jnp.dot(vT * es, bd, preferred_element_type=f32)
    den = jnp.dot(es, bd, preferred_element_type=f32)
    pT = num / jnp.maximum(den, 1e-30)
    i = lax.broadcasted_iota(jnp.int32, (16, 16), 0)
    j = lax.broadcasted_iota(jnp.int32, (16, 16), 1)
    eye = (i == j).astype(f32)
    pe = lax.dot_general(eye, pT, (((1,), (1,)), ((), ())),
                         preferred_element_type=f32)
    h1 = jnp.maximum(
        jnp.dot(pe, w1_ref[...], preferred_element_type=f32) + b1_ref[...],
        0.0)
    h2 = jnp.maximum(
        jnp.dot(h1, w2_ref[...], preferred_element_type=f32) + b2_ref[...],
        0.0)
    out_ref[...] = (
        jnp.dot(h2, w3_ref[...], preferred_element_type=f32) + b3_ref[...])


def kernel(edge_index, edge_weight, codes_x, node_emb, gcn_W1, gcn_b1,
           gcn_W2, gcn_b2, attn_code, attn_visit, dec_W1, dec_b1,
           dec_W2, dec_b2, dec_W3, dec_b3):
    f32 = jnp.float32
    pad = EPAD - E
    spread = (jnp.arange(pad, dtype=jnp.int32) * 131) % N_NODES
    srcp = jnp.concatenate([edge_index[0], spread]).reshape(NW, ENCH, ECH)
    dstp = jnp.concatenate([edge_index[1], spread]).reshape(NW, ENCH, ECH)
    wp = jnp.concatenate([edge_weight, jnp.zeros((pad,), f32)]
                         ).reshape(NW, ENCH, ECH)

    p10, p11 = _sc_segsum(node_emb, srcp, dstp, wp)
    h = _mm2_relu(p10, p11, gcn_W1, gcn_b1, 2048)

    p20, p21 = _sc_segsum(h, srcp, dstp, wp)
    a2 = p20 + p21

    codes = codes_x.reshape(NW, CODE_NCH, CH)
    g = _sc_code_gather(a2, codes)

    codesf = codes_x.astype(f32).reshape(C_CODES // CB, 1, CB)
    visit_T = pl.pallas_call(
        _code_attn_body,
        grid=(C_CODES // CB,),
        in_specs=[
            pl.BlockSpec((CB, D_IN), lambda i: (i, 0)),
            pl.BlockSpec((D_IN, D_OUT), lambda i: (0, 0)),
            pl.BlockSpec((1, D_OUT), lambda i: (0, 0)),
            pl.BlockSpec((1, D_OUT), lambda i: (0, 0)),
            pl.BlockSpec((1, 1, CB), lambda i: (i, 0, 0)),
        ],
        out_specs=pl.BlockSpec((D_OUT, CB // 32), lambda i: (0, i)),
        out_shape=jax.ShapeDtypeStruct((D_OUT, 1024), f32),
    )(g, gcn_W2, gcn_b2.reshape(1, -1), attn_code, codesf)

    vmask = jnp.any(codes_x > 0, axis=-1).astype(f32).reshape(1, 1024)
    w3p = jnp.zeros((dec_W3.shape[0], 1024), f32).at[:, :N_CLS].set(dec_W3)
    b3p = jnp.zeros((1, 1024), f32).at[0, :N_CLS].set(dec_b3)
    out = pl.pallas_call(
        _visit_attn_decoder_body,
        out_shape=jax.ShapeDtypeStruct((16, 1024), f32),
    )(visit_T, vmask, attn_visit, dec_W1, dec_b1.reshape(1, -1), dec_W2,
      dec_b2.reshape(1, -1), w3p, b3p)
    return out[:, :N_CLS]
